# baseline (speedup 1.0000x reference)
import functools

import jax
import jax.numpy as jnp
from jax import lax
from jax.experimental import pallas as pl
from jax.experimental.pallas import tpu as pltpu
from jax.experimental.pallas import tpu_sc as plsc

B = 16384
ND = 16
GRP = 128 // ND
NC = 2
NS = 16
NW = NC * NS
BPW = B // NW
CHUNK = 128
NCHUNK = BPW // CHUNK
BLKS = CHUNK // ND


def _sc_body(user_hbm, item_hbm, uids_hbm, pids_hbm, nids_hbm, d_hbm,
             idx_u, idx_p, idx_n, cu, cp, cn, u_sl, p_sl, n_sl, d_v, sem):
    wid = lax.axis_index("s") * NC + lax.axis_index("c")
    base = wid * BPW

    pltpu.sync_copy(uids_hbm.at[pl.ds(base, BPW)], idx_u)
    pltpu.sync_copy(pids_hbm.at[pl.ds(base, BPW)], idx_p)
    pltpu.sync_copy(nids_hbm.at[pl.ds(base, BPW)], idx_n)

    lane = lax.iota(jnp.int32, ND)

    for c in range(NCHUNK):
        for v in range(CHUNK // ND):
            src = pl.ds(c * CHUNK + v * ND, ND)
            dst = pl.ds(v * ND, ND)
            cu[dst] = lax.shift_right_logical(idx_u[src], GRP.bit_length() - 1)
            cp[dst] = lax.shift_right_logical(idx_p[src], GRP.bit_length() - 1)
            cn[dst] = lax.shift_right_logical(idx_n[src], GRP.bit_length() - 1)

        a = pltpu.async_copy(user_hbm.at[cu], u_sl, sem)
        b_ = pltpu.async_copy(item_hbm.at[cp], p_sl, sem)
        cc = pltpu.async_copy(item_hbm.at[cn], n_sl, sem)
        a.wait()
        b_.wait()
        cc.wait()

        def blk(b, _):
            rows = b * ND + lane
            sl16 = pl.ds(c * CHUNK + b * ND, ND)
            ou = (idx_u[sl16] & (GRP - 1)) * ND
            op = (idx_p[sl16] & (GRP - 1)) * ND
            on = (idx_n[sl16] & (GRP - 1)) * ND
            acc = jnp.zeros((ND,), jnp.float32)
            for j in range(ND):
                u = plsc.load_gather(u_sl, [rows, ou + j])
                p = plsc.load_gather(p_sl, [rows, op + j])
                n = plsc.load_gather(n_sl, [rows, on + j])
                acc = acc + u * (p - n)
            d_v[pl.ds(c * CHUNK + b * ND, ND)] = acc
            return ()

        lax.fori_loop(0, BLKS, blk, (), unroll=False)

    pltpu.sync_copy(d_v, d_hbm.at[pl.ds(base, BPW)])


@jax.jit
def _sc_scores(user_g, item_g, uids, pids, nids):
    mesh = plsc.VectorSubcoreMesh(core_axis_name="c", subcore_axis_name="s")
    kfn = pl.kernel(
        _sc_body,
        out_type=jax.ShapeDtypeStruct((B,), jnp.float32),
        mesh=mesh,
        scratch_types=[
            pltpu.VMEM((BPW,), jnp.int32),
            pltpu.VMEM((BPW,), jnp.int32),
            pltpu.VMEM((BPW,), jnp.int32),
            pltpu.VMEM((CHUNK,), jnp.int32),
            pltpu.VMEM((CHUNK,), jnp.int32),
            pltpu.VMEM((CHUNK,), jnp.int32),
            pltpu.VMEM((CHUNK, 128), jnp.float32),
            pltpu.VMEM((CHUNK, 128), jnp.float32),
            pltpu.VMEM((CHUNK, 128), jnp.float32),
            pltpu.VMEM((BPW,), jnp.float32),
            pltpu.SemaphoreType.DMA,
        ],
        compiler_params=pltpu.CompilerParams(needs_layout_passes=False),
    )
    return kfn(user_g, item_g, uids, pids, nids)


def _loss_body(d_ref, out_ref):
    d = d_ref[...]
    loss = jnp.maximum(-d, 0.0) + jnp.log1p(jnp.exp(-jnp.abs(d)))
    out_ref[0, 0] = jnp.sum(loss) * (1.0 / B)


@jax.jit
def _tc_loss(d):
    return pl.pallas_call(
        _loss_body,
        out_shape=jax.ShapeDtypeStruct((1, 1), jnp.float32),
        out_specs=pl.BlockSpec(memory_space=pltpu.SMEM),
    )(d)


def kernel(X, user_emb, item_emb):
    uids = X[:, 0]
    pids = X[:, 1]
    nids = X[:, 2]
    d = _sc_scores(user_emb.reshape(-1, 128), item_emb.reshape(-1, 128),
                   uids, pids, nids)
    loss = _tc_loss(d.reshape(128, 128))
    return loss.reshape(())


# speedup vs baseline: 1.4994x; 1.4994x over previous
import functools

import jax
import jax.numpy as jnp
from jax import lax
from jax.experimental import pallas as pl
from jax.experimental.pallas import tpu as pltpu
from jax.experimental.pallas import tpu_sc as plsc

B = 16384
ND = 16
NC = 2
NS = 16
NW = NC * NS
BPW = B // NW
CHUNK = 128
NCHUNK = BPW // CHUNK


def _sc_body(user_hbm, item_hbm, uids_hbm, pids_hbm, nids_hbm, d_hbm,
             idx_u, idx_p, idx_n, u_buf, p_buf, n_buf, d_v, sem):
    wid = lax.axis_index("s") * NC + lax.axis_index("c")
    base = wid * BPW

    pltpu.sync_copy(uids_hbm.at[pl.ds(base, BPW)], idx_u)
    pltpu.sync_copy(pids_hbm.at[pl.ds(base, BPW)], idx_p)
    pltpu.sync_copy(nids_hbm.at[pl.ds(base, BPW)], idx_n)

    lane = lax.iota(jnp.int32, ND)

    for c in range(NCHUNK):
        def fire(b, _):
            sl16 = pl.ds(c * CHUNK + b * ND, ND)
            ids_u = idx_u[sl16]
            ids_p = idx_p[sl16]
            ids_n = idx_n[sl16]
            for k in range(ND):
                i = b * ND + k
                pltpu.async_copy(user_hbm.at[pl.ds(ids_u[k], 1)],
                                 u_buf.at[pl.ds(i, 1)], sem)
                pltpu.async_copy(item_hbm.at[pl.ds(ids_p[k], 1)],
                                 p_buf.at[pl.ds(i, 1)], sem)
                pltpu.async_copy(item_hbm.at[pl.ds(ids_n[k], 1)],
                                 n_buf.at[pl.ds(i, 1)], sem)
            return ()

        lax.fori_loop(0, CHUNK // ND, fire, (), unroll=False)

        def drain(i, _):
            pltpu.make_async_copy(user_hbm.at[pl.ds(0, 1)],
                                  u_buf.at[pl.ds(i, 1)], sem).wait()
            pltpu.make_async_copy(item_hbm.at[pl.ds(0, 1)],
                                  p_buf.at[pl.ds(i, 1)], sem).wait()
            pltpu.make_async_copy(item_hbm.at[pl.ds(0, 1)],
                                  n_buf.at[pl.ds(i, 1)], sem).wait()
            return ()

        lax.fori_loop(0, CHUNK, drain, (), unroll=False)

        def blk(b, _):
            acc = jnp.zeros((ND,), jnp.float32)
            for k in range(ND):
                i = b * ND + k
                t = u_buf[i] * (p_buf[i] - n_buf[i])
                dk = jnp.sum(t)
                acc = jnp.where(lane == k, dk, acc)
            d_v[pl.ds(c * CHUNK + b * ND, ND)] = acc
            return ()

        lax.fori_loop(0, CHUNK // ND, blk, (), unroll=False)

    pltpu.sync_copy(d_v, d_hbm.at[pl.ds(base, BPW)])


@jax.jit
def _sc_scores(user_emb, item_emb, uids, pids, nids):
    mesh = plsc.VectorSubcoreMesh(core_axis_name="c", subcore_axis_name="s")
    kfn = pl.kernel(
        _sc_body,
        out_type=jax.ShapeDtypeStruct((B,), jnp.float32),
        mesh=mesh,
        scratch_types=[
            pltpu.VMEM((BPW,), jnp.int32),
            pltpu.VMEM((BPW,), jnp.int32),
            pltpu.VMEM((BPW,), jnp.int32),
            pltpu.VMEM((CHUNK, ND), jnp.float32),
            pltpu.VMEM((CHUNK, ND), jnp.float32),
            pltpu.VMEM((CHUNK, ND), jnp.float32),
            pltpu.VMEM((BPW,), jnp.float32),
            pltpu.SemaphoreType.DMA,
        ],
        compiler_params=pltpu.CompilerParams(needs_layout_passes=False),
    )
    return kfn(user_emb, item_emb, uids, pids, nids)


def _loss_body(d_ref, out_ref):
    d = d_ref[...]
    loss = jnp.maximum(-d, 0.0) + jnp.log1p(jnp.exp(-jnp.abs(d)))
    out_ref[0, 0] = jnp.sum(loss) * (1.0 / B)


@jax.jit
def _tc_loss(d):
    return pl.pallas_call(
        _loss_body,
        out_shape=jax.ShapeDtypeStruct((1, 1), jnp.float32),
        out_specs=pl.BlockSpec(memory_space=pltpu.SMEM),
    )(d)


def kernel(X, user_emb, item_emb):
    uids = X[:, 0]
    pids = X[:, 1]
    nids = X[:, 2]
    d = _sc_scores(user_emb, item_emb, uids, pids, nids)
    loss = _tc_loss(d.reshape(128, 128))
    return loss.reshape(())


# speedup vs baseline: 4.4303x; 2.9548x over previous
import functools

import jax
import jax.numpy as jnp
from jax import lax
from jax.experimental import pallas as pl
from jax.experimental.pallas import tpu as pltpu
from jax.experimental.pallas import tpu_sc as plsc

B = 16384
ND = 16
NC = 2
NS = 16
NW = NC * NS
BPW = B // NW
WAVE = 16
NWAVE = BPW // WAVE
TILE = 128


def _softplus_neg(d):
    q = jnp.exp(-jnp.abs(d))
    t = q / (2.0 + q)
    t2 = t * t
    poly = 1.0 + t2 * (
        (1.0 / 3.0) + t2 * ((1.0 / 5.0) + t2 * ((1.0 / 7.0) + t2 * (1.0 / 9.0)))
    )
    return jnp.maximum(-d, 0.0) + 2.0 * t * poly


def _sc_body(user_t, item_t, uids_hbm, pids_hbm, nids_hbm, part_hbm,
             idx_u, idx_p, idx_n, u_blk, p_blk, n_blk, s_v, sem):
    wid = lax.axis_index("s") * NC + lax.axis_index("c")
    base = wid * BPW

    pltpu.sync_copy(uids_hbm.at[pl.ds(base, BPW)], idx_u)
    pltpu.sync_copy(pids_hbm.at[pl.ds(base, BPW)], idx_p)
    pltpu.sync_copy(nids_hbm.at[pl.ds(base, BPW)], idx_n)

    lane = lax.iota(jnp.int32, ND)

    def wave(w, s_acc):
        sl16 = pl.ds(w * WAVE, WAVE)
        ids_u = idx_u[sl16]
        ids_p = idx_p[sl16]
        ids_n = idx_n[sl16]
        blk_u = lax.shift_left(lax.shift_right_logical(ids_u, 7), 7)
        blk_p = lax.shift_left(lax.shift_right_logical(ids_p, 7), 7)
        blk_n = lax.shift_left(lax.shift_right_logical(ids_n, 7), 7)
        sub_u = ids_u & (TILE - 1)
        sub_p = ids_p & (TILE - 1)
        sub_n = ids_n & (TILE - 1)

        for k in range(WAVE):
            dsl = pl.ds(k * ND, ND)
            pltpu.async_copy(
                user_t.at[:, pl.ds(pl.multiple_of(blk_u[k], TILE), TILE)],
                u_blk.at[dsl, :], sem)
            pltpu.async_copy(
                item_t.at[:, pl.ds(pl.multiple_of(blk_p[k], TILE), TILE)],
                p_blk.at[dsl, :], sem)
            pltpu.async_copy(
                item_t.at[:, pl.ds(pl.multiple_of(blk_n[k], TILE), TILE)],
                n_blk.at[dsl, :], sem)
        for k in range(WAVE):
            dsl = pl.ds(k * ND, ND)
            pltpu.make_async_copy(user_t.at[:, pl.ds(0, TILE)],
                                  u_blk.at[dsl, :], sem).wait()
            pltpu.make_async_copy(item_t.at[:, pl.ds(0, TILE)],
                                  p_blk.at[dsl, :], sem).wait()
            pltpu.make_async_copy(item_t.at[:, pl.ds(0, TILE)],
                                  n_blk.at[dsl, :], sem).wait()

        d = jnp.zeros((ND,), jnp.float32)
        for k in range(WAVE):
            rows = k * ND + lane
            u = plsc.load_gather(u_blk, [rows, jnp.full((ND,), sub_u[k])])
            p = plsc.load_gather(p_blk, [rows, jnp.full((ND,), sub_p[k])])
            n = plsc.load_gather(n_blk, [rows, jnp.full((ND,), sub_n[k])])
            d = jnp.where(lane == k, jnp.sum(u * (p - n)), d)
        return s_acc + _softplus_neg(d)

    s_acc = lax.fori_loop(0, NWAVE, wave, jnp.zeros((ND,), jnp.float32),
                          unroll=False)
    s_v[...] = s_acc
    pltpu.sync_copy(s_v, part_hbm.at[wid])


@jax.jit
def _sc_partials(user_emb, item_emb, uids, pids, nids):
    user_t = user_emb.T
    item_t = item_emb.T
    mesh = plsc.VectorSubcoreMesh(core_axis_name="c", subcore_axis_name="s")
    kfn = pl.kernel(
        _sc_body,
        out_type=jax.ShapeDtypeStruct((NW, ND), jnp.float32),
        mesh=mesh,
        scratch_types=[
            pltpu.VMEM((BPW,), jnp.int32),
            pltpu.VMEM((BPW,), jnp.int32),
            pltpu.VMEM((BPW,), jnp.int32),
            pltpu.VMEM((WAVE * ND, TILE), jnp.float32),
            pltpu.VMEM((WAVE * ND, TILE), jnp.float32),
            pltpu.VMEM((WAVE * ND, TILE), jnp.float32),
            pltpu.VMEM((ND,), jnp.float32),
            pltpu.SemaphoreType.DMA,
        ],
        compiler_params=pltpu.CompilerParams(needs_layout_passes=False),
    )
    return kfn(user_t, item_t, uids, pids, nids)


def _loss_body(part_ref, out_ref):
    out_ref[0, 0] = jnp.sum(part_ref[...]) * (1.0 / B)


@jax.jit
def _tc_loss(part):
    return pl.pallas_call(
        _loss_body,
        out_shape=jax.ShapeDtypeStruct((1, 1), jnp.float32),
        out_specs=pl.BlockSpec(memory_space=pltpu.SMEM),
    )(part)


def kernel(X, user_emb, item_emb):
    uids = X[:, 0]
    pids = X[:, 1]
    nids = X[:, 2]
    part = _sc_partials(user_emb, item_emb, uids, pids, nids)
    loss = _tc_loss(part)
    return loss.reshape(())
